# baseline (speedup 1.0000x reference)
import jax
import jax.numpy as jnp
from jax import lax
from jax.experimental import pallas as pl
from jax.experimental.pallas import tpu as pltpu
from jax.experimental.pallas import tpu_sc as plsc

_B, _S, _D = 4, 4096, 1024
_NW = 32
_SPW = _S // _NW
_K = 32
_NCH = _SPW // _K
_NSTEP = _NCH * _B


def _emb_body(tok_ref, emb_ref, pos_ref, out_ref, idx_v, pos_v, emb_v,
              gsem, osem):
    cid = lax.axis_index("core")
    sid = lax.axis_index("subcore")
    wid = sid * 2 + cid
    s_base = wid * _SPW

    pltpu.sync_copy(tok_ref.at[wid], idx_v)

    def gather_start(t, par):
        pltpu.async_copy(emb_ref.at[idx_v.at[t]], emb_v.at[par], gsem)

    def gather_wait(t, par):
        pltpu.make_async_copy(emb_ref.at[idx_v.at[t]], emb_v.at[par],
                              gsem).wait()

    def out_start(b, s0, par):
        pltpu.async_copy(emb_v.at[par], out_ref.at[b, pl.ds(s0, _K)], osem)

    def out_wait(par):
        pltpu.make_async_copy(emb_v.at[par], out_ref.at[0, pl.ds(0, _K)],
                              osem).wait()

    def add_pos(par):
        def row(r, carry):
            for j in range(_D // 16):
                sl = pl.ds(j * 16, 16)
                emb_v[par, r, sl] = emb_v[par, r, sl] + pos_v[r, sl]
            return carry
        lax.fori_loop(0, _K, row, 0)

    pltpu.sync_copy(pos_ref.at[pl.ds(s_base, _K)], pos_v)
    gather_start(0, 0)
    for b in range(_B):
        par, npar = b % 2, (b + 1) % 2
        gather_wait(b, par)
        if b > 0:
            out_wait(npar)
        gather_start(b + 1, npar)
        add_pos(par)
        out_start(b, s_base, par)

    def chunk(c, carry):
        s0 = s_base + c * _K
        t0 = c * _B
        pltpu.sync_copy(pos_ref.at[pl.ds(s0, _K)], pos_v)
        for b in range(_B):
            par, npar = b % 2, (b + 1) % 2
            gather_wait(t0 + b, par)
            out_wait(npar)
            gather_start(t0 + b + 1, npar)
            add_pos(par)
            out_start(b, s0, par)
        return carry

    lax.fori_loop(1, _NCH, chunk, 0)

    gather_wait(_NSTEP, 0)
    out_wait(1)


def kernel(tokens, emb_table, pos_table):
    tok = (tokens.astype(jnp.int32)
           .reshape(_B, _NW, _NCH, _K)
           .transpose(1, 2, 0, 3)
           .reshape(_NW, _NSTEP, _K))
    tok = jnp.pad(tok, ((0, 0), (0, 1), (0, 0)))
    mesh = plsc.VectorSubcoreMesh(core_axis_name="core",
                                  subcore_axis_name="subcore")
    f = pl.kernel(
        _emb_body,
        out_type=jax.ShapeDtypeStruct((_B, _S, _D), jnp.float32),
        mesh=mesh,
        scratch_types=[
            pltpu.VMEM((_NSTEP + 1, _K), jnp.int32),
            pltpu.VMEM((_K, _D), jnp.float32),
            pltpu.VMEM((2, _K, _D), jnp.float32),
            pltpu.SemaphoreType.DMA,
            pltpu.SemaphoreType.DMA,
        ],
    )
    return f(tok, emb_table, pos_table)


# speedup vs baseline: 1.3034x; 1.3034x over previous
import jax
import jax.numpy as jnp
from jax import lax
from jax.experimental import pallas as pl
from jax.experimental.pallas import tpu as pltpu
from jax.experimental.pallas import tpu_sc as plsc

_B, _S, _D = 4, 4096, 1024
_NW = 32
_SPW = _S // _NW
_K = 16
_NCH = _SPW // _K


def _emb_body(tok_ref, emb_ref, pos_ref, out_ref, idx_v, pos_v, emb_v,
              gsem0, gsem1, osem):
    gsems = (gsem0, gsem1)
    cid = lax.axis_index("core")
    sid = lax.axis_index("subcore")
    wid = sid * 2 + cid
    s_base = wid * _SPW

    pltpu.sync_copy(tok_ref.at[wid], idx_v)

    def add_pos(par):
        def row(r, carry):
            for j in range(_D // 16):
                sl = pl.ds(j * 16, 16)
                emb_v[par, r, sl] = emb_v[par, r, sl] + pos_v[r, sl]
            return carry
        lax.fori_loop(0, _K, row, 0)

    def chunk(c, carry):
        s0 = s_base + c * _K
        pltpu.sync_copy(pos_ref.at[pl.ds(s0, _K)], pos_v)
        g = pltpu.async_copy(emb_ref.at[idx_v.at[0, c]], emb_v.at[0], gsems[0])
        outs = []
        for b in range(_B):
            g_next = None
            if b + 1 < _B:
                g_next = pltpu.async_copy(emb_ref.at[idx_v.at[b + 1, c]],
                                          emb_v.at[b + 1],
                                          gsems[(b + 1) % 2])
            g.wait()
            add_pos(b)
            outs.append(pltpu.async_copy(emb_v.at[b],
                                         out_ref.at[b, pl.ds(s0, _K)], osem))
            g = g_next
        for o in outs:
            o.wait()
        return carry

    lax.fori_loop(0, _NCH, chunk, 0)


def kernel(tokens, emb_table, pos_table):
    tok = (tokens.astype(jnp.int32)
           .reshape(_B, _NW, _NCH, _K)
           .transpose(1, 0, 2, 3))
    mesh = plsc.VectorSubcoreMesh(core_axis_name="core",
                                  subcore_axis_name="subcore")
    f = pl.kernel(
        _emb_body,
        out_type=jax.ShapeDtypeStruct((_B, _S, _D), jnp.float32),
        mesh=mesh,
        scratch_types=[
            pltpu.VMEM((_B, _NCH, _K), jnp.int32),
            pltpu.VMEM((_K, _D), jnp.float32),
            pltpu.VMEM((_B, _K, _D), jnp.float32),
            pltpu.SemaphoreType.DMA,
            pltpu.SemaphoreType.DMA,
            pltpu.SemaphoreType.DMA,
        ],
    )
    return f(tok, emb_table, pos_table)


# speedup vs baseline: 1.3416x; 1.0293x over previous
import jax
import jax.numpy as jnp
from jax import lax
from jax.experimental import pallas as pl
from jax.experimental.pallas import tpu as pltpu
from jax.experimental.pallas import tpu_sc as plsc

_B, _S, _D = 4, 4096, 1024
_NW = 32
_SPW = _S // _NW
_K = 16
_NCH = _SPW // _K


def _emb_body(tok_ref, emb_ref, pos_ref, out_ref, idx_v, pos_v, emb_v,
              gsem0, gsem1, gsem2, gsem3, osem):
    gsems = (gsem0, gsem1, gsem2, gsem3)
    cid = lax.axis_index("core")
    sid = lax.axis_index("subcore")
    wid = sid * 2 + cid
    s_base = wid * _SPW

    pltpu.sync_copy(tok_ref.at[wid], idx_v)

    def add_pos(par):
        def row(r, carry):
            for j in range(_D // 16):
                sl = pl.ds(j * 16, 16)
                emb_v[par, r, sl] = emb_v[par, r, sl] + pos_v[r, sl]
            return carry
        lax.fori_loop(0, _K, row, 0)

    def gather(b, c):
        return pltpu.async_copy(emb_ref.at[idx_v.at[b, c]], emb_v.at[b],
                                gsems[b])

    def chunk(c, carry):
        s0 = s_base + c * _K
        gs = [gather(0, c), gather(1, c), None, None]
        pltpu.sync_copy(pos_ref.at[pl.ds(s0, _K)], pos_v)
        outs = []
        for b in range(_B):
            gs[b].wait()
            if b + 2 < _B:
                gs[b + 2] = gather(b + 2, c)
            add_pos(b)
            outs.append(pltpu.async_copy(emb_v.at[b],
                                         out_ref.at[b, pl.ds(s0, _K)], osem))
        for o in outs:
            o.wait()
        return carry

    lax.fori_loop(0, _NCH, chunk, 0)


def kernel(tokens, emb_table, pos_table):
    tok = (tokens.astype(jnp.int32)
           .reshape(_B, _NW, _NCH, _K)
           .transpose(1, 0, 2, 3))
    mesh = plsc.VectorSubcoreMesh(core_axis_name="core",
                                  subcore_axis_name="subcore")
    f = pl.kernel(
        _emb_body,
        out_type=jax.ShapeDtypeStruct((_B, _S, _D), jnp.float32),
        mesh=mesh,
        scratch_types=[
            pltpu.VMEM((_B, _NCH, _K), jnp.int32),
            pltpu.VMEM((_K, _D), jnp.float32),
            pltpu.VMEM((_B, _K, _D), jnp.float32),
            pltpu.SemaphoreType.DMA,
            pltpu.SemaphoreType.DMA,
            pltpu.SemaphoreType.DMA,
            pltpu.SemaphoreType.DMA,
            pltpu.SemaphoreType.DMA,
        ],
    )
    return f(tok, emb_table, pos_table)
